# baseline (speedup 1.0000x reference)
import functools

import jax
import jax.numpy as jnp
from jax.experimental import pallas as pl
from jax.experimental.pallas import tpu as pltpu


def _erf(x):
    p = jnp.float32(0.3275911)
    a1 = jnp.float32(0.254829592)
    a2 = jnp.float32(-0.284496736)
    a3 = jnp.float32(1.421413741)
    a4 = jnp.float32(-1.453152027)
    a5 = jnp.float32(1.061405429)
    ax = jnp.abs(x)
    t = 1.0 / (1.0 + p * ax)
    poly = t * (a1 + t * (a2 + t * (a3 + t * (a4 + t * a5))))
    y = 1.0 - poly * jnp.exp(-ax * ax)
    return jnp.sign(x) * y


def _gelu(x):
    return x * 0.5 * (1.0 + _erf(x * jnp.float32(0.7071067811865476)))


def _mlp_kernel(x_ref, ids_ref, w1_ref, b1_ref, w2_ref, b2_ref, w3_ref, b3_ref,
                ds_ref, do_ref, a_ref, b_ref, u_ref, v_ref):
    S = x_ref.shape[0]
    E = w1_ref.shape[0]
    H = ds_ref.shape[1]
    x = x_ref[...]
    ids = ids_ref[...]
    eiota = jax.lax.broadcasted_iota(jnp.int32, (S, E), 1)
    onehot = (ids == eiota).astype(jnp.float32)

    def dot_t(a, w):
        return jax.lax.dot_general(a, w, (((1,), (1,)), ((), ())),
                                   preferred_element_type=jnp.float32)

    pb = jnp.zeros((S, H), jnp.float32)
    for e in range(E):
        h1 = _gelu(dot_t(x, w1_ref[e]) + b1_ref[e])
        h2 = _gelu(dot_t(h1, w2_ref[e]) + b2_ref[e])
        eo = dot_t(h2, w3_ref[e]) + b3_ref[e]
        pb = pb + onehot[:, e:e + 1] * eo

    counts = jnp.sum(onehot, axis=0, keepdims=True)
    inv_s = jnp.float32(1.0 / S)
    c = jnp.dot(counts, ds_ref[...], preferred_element_type=jnp.float32) \
        * (inv_s * jnp.float32(0.01))
    off = jnp.dot(counts, do_ref[...], preferred_element_type=jnp.float32) * inv_s
    eoff = jnp.exp(off)

    kk = jax.lax.broadcasted_iota(jnp.int32, (S, 1), 0).astype(jnp.float32) \
        - jnp.float32(S // 2)
    u = jnp.exp(-kk * c)
    v = jnp.exp(kk * c)
    pbo = pb * eoff
    a_ref[...] = pbo * u
    b_ref[...] = pbo * v
    u_ref[...] = u
    v_ref[...] = v


def _bias_kernel(a_ref, b_ref, u_ref, v_ref, o_ref, *, tile_i, tile_j):
    ib = pl.program_id(1)
    jb = pl.program_id(2)
    i0 = ib * tile_i
    j0 = jb * tile_j
    below = i0 >= j0 + tile_j
    above = j0 >= i0 + tile_i

    @pl.when(below)
    def _():
        o_ref[0] = a_ref[0] * v_ref[0]

    @pl.when(above)
    def _():
        o_ref[0] = b_ref[0] * u_ref[0]

    @pl.when(jnp.logical_not(jnp.logical_or(below, above)))
    def _():
        rows = jax.lax.broadcasted_iota(jnp.int32, (tile_i, tile_j), 0) + i0
        cols = jax.lax.broadcasted_iota(jnp.int32, (tile_i, tile_j), 1) + j0
        o_ref[0] = jnp.where(rows >= cols, a_ref[0] * v_ref[0],
                             b_ref[0] * u_ref[0])


def kernel(scalars, seq_len, inscription_ids, W1, b1, W2, b2, W3, b3,
           distance_scales, distance_offsets):
    del seq_len
    B, S, D = scalars.shape
    H = W3.shape[1]

    x = scalars.reshape(S, D)
    ids = inscription_ids.reshape(S, 1).astype(jnp.int32)

    sh = jax.ShapeDtypeStruct((S, H), jnp.float32)
    a2, b2_, u2, v2 = pl.pallas_call(
        _mlp_kernel,
        out_shape=(sh, sh, sh, sh),
    )(x, ids, W1, b1, W2, b2, W3, b3, distance_scales, distance_offsets)

    a3 = a2.T.reshape(H, S, 1)
    b3_ = b2_.T.reshape(H, S, 1)
    u3 = u2.T.reshape(H, 1, S)
    v3 = v2.T.reshape(H, 1, S)

    TILE_I = 256
    TILE_J = 512
    grid = (H, S // TILE_I, S // TILE_J)
    row_spec = pl.BlockSpec((1, TILE_I, 1), lambda h, i, j: (h, i, 0))
    col_spec = pl.BlockSpec((1, 1, TILE_J), lambda h, i, j: (h, 0, j))
    bias = pl.pallas_call(
        functools.partial(_bias_kernel, tile_i=TILE_I, tile_j=TILE_J),
        grid=grid,
        in_specs=[row_spec, row_spec, col_spec, col_spec],
        out_specs=pl.BlockSpec((1, TILE_I, TILE_J), lambda h, i, j: (h, i, j)),
        out_shape=jax.ShapeDtypeStruct((H, S, S), jnp.float32),
    )(a3, b3_, u3, v3)

    return bias.reshape(B, H, S, S)


# speedup vs baseline: 2.1838x; 2.1838x over previous
import functools

import jax
import jax.numpy as jnp
from jax.experimental import pallas as pl
from jax.experimental.pallas import tpu as pltpu


def _erf(x):
    p = jnp.float32(0.3275911)
    a1 = jnp.float32(0.254829592)
    a2 = jnp.float32(-0.284496736)
    a3 = jnp.float32(1.421413741)
    a4 = jnp.float32(-1.453152027)
    a5 = jnp.float32(1.061405429)
    ax = jnp.abs(x)
    t = 1.0 / (1.0 + p * ax)
    poly = t * (a1 + t * (a2 + t * (a3 + t * (a4 + t * a5))))
    y = 1.0 - poly * jnp.exp(-ax * ax)
    return jnp.sign(x) * y


def _gelu(x):
    return x * 0.5 * (1.0 + _erf(x * jnp.float32(0.7071067811865476)))


def _mlp_kernel(x_ref, ids_ref, w1_ref, b1_ref, w2_ref, b2_ref, w3_ref, b3_ref,
                ds_ref, do_ref, a_ref, b_ref, u_ref, v_ref):
    S = x_ref.shape[0]
    E = w1_ref.shape[0]
    H = ds_ref.shape[1]
    x = x_ref[...]
    ids = ids_ref[...]
    eiota = jax.lax.broadcasted_iota(jnp.int32, (S, E), 1)
    onehot = (ids == eiota).astype(jnp.float32)

    def dot_t(a, w):
        return jax.lax.dot_general(a, w, (((1,), (1,)), ((), ())),
                                   preferred_element_type=jnp.float32)

    pb = jnp.zeros((S, H), jnp.float32)
    for e in range(E):
        h1 = _gelu(dot_t(x, w1_ref[e]) + b1_ref[e])
        h2 = _gelu(dot_t(h1, w2_ref[e]) + b2_ref[e])
        eo = dot_t(h2, w3_ref[e]) + b3_ref[e]
        pb = pb + onehot[:, e:e + 1] * eo

    counts = jnp.sum(onehot, axis=0, keepdims=True)
    inv_s = jnp.float32(1.0 / S)
    c = jnp.dot(counts, ds_ref[...], preferred_element_type=jnp.float32) \
        * (inv_s * jnp.float32(0.01))
    off = jnp.dot(counts, do_ref[...], preferred_element_type=jnp.float32) * inv_s
    eoff = jnp.exp(off)

    kk = jax.lax.broadcasted_iota(jnp.int32, (S, 1), 0).astype(jnp.float32) \
        - jnp.float32(S // 2)
    u = jnp.exp(-kk * c)
    v = jnp.exp(kk * c)
    pbo = pb * eoff
    a_ref[...] = pbo * u
    b_ref[...] = pbo * v
    u_ref[...] = u
    v_ref[...] = v


def _bias_kernel(a_ref, b_ref, u_ref, v_ref, o_ref, *, tile_i, tile_j):
    ib = pl.program_id(1)
    jb = pl.program_id(2)
    i0 = ib * tile_i
    j0 = jb * tile_j
    below = i0 >= j0 + tile_j
    above = j0 >= i0 + tile_i

    @pl.when(below)
    def _():
        o_ref[0] = a_ref[0] * v_ref[0]

    @pl.when(above)
    def _():
        o_ref[0] = b_ref[0] * u_ref[0]

    @pl.when(jnp.logical_not(jnp.logical_or(below, above)))
    def _():
        rows = jax.lax.broadcasted_iota(jnp.int32, (tile_i, tile_j), 0) + i0
        cols = jax.lax.broadcasted_iota(jnp.int32, (tile_i, tile_j), 1) + j0
        o_ref[0] = jnp.where(rows >= cols, a_ref[0] * v_ref[0],
                             b_ref[0] * u_ref[0])


def kernel(scalars, seq_len, inscription_ids, W1, b1, W2, b2, W3, b3,
           distance_scales, distance_offsets):
    del seq_len
    B, S, D = scalars.shape
    H = W3.shape[1]

    x = scalars.reshape(S, D)
    ids = inscription_ids.reshape(S, 1).astype(jnp.int32)

    sh = jax.ShapeDtypeStruct((S, H), jnp.float32)
    a2, b2_, u2, v2 = pl.pallas_call(
        _mlp_kernel,
        out_shape=(sh, sh, sh, sh),
    )(x, ids, W1, b1, W2, b2, W3, b3, distance_scales, distance_offsets)

    a3 = a2.T.reshape(H, S, 1)
    b3_ = b2_.T.reshape(H, S, 1)
    u3 = u2.T.reshape(H, 1, S)
    v3 = v2.T.reshape(H, 1, S)

    TILE_I = 512
    TILE_J = S
    grid = (H, S // TILE_I, S // TILE_J)
    row_spec = pl.BlockSpec((1, TILE_I, 1), lambda h, i, j: (h, i, 0))
    col_spec = pl.BlockSpec((1, 1, TILE_J), lambda h, i, j: (h, 0, j))
    bias = pl.pallas_call(
        functools.partial(_bias_kernel, tile_i=TILE_I, tile_j=TILE_J),
        grid=grid,
        in_specs=[row_spec, row_spec, col_spec, col_spec],
        out_specs=pl.BlockSpec((1, TILE_I, TILE_J), lambda h, i, j: (h, i, j)),
        out_shape=jax.ShapeDtypeStruct((H, S, S), jnp.float32),
    )(a3, b3_, u3, v3)

    return bias.reshape(B, H, S, S)


# speedup vs baseline: 2.3780x; 1.0889x over previous
import functools

import jax
import jax.numpy as jnp
from jax.experimental import pallas as pl
from jax.experimental.pallas import tpu as pltpu


def _erf(x):
    p = jnp.float32(0.3275911)
    a1 = jnp.float32(0.254829592)
    a2 = jnp.float32(-0.284496736)
    a3 = jnp.float32(1.421413741)
    a4 = jnp.float32(-1.453152027)
    a5 = jnp.float32(1.061405429)
    ax = jnp.abs(x)
    t = 1.0 / (1.0 + p * ax)
    poly = t * (a1 + t * (a2 + t * (a3 + t * (a4 + t * a5))))
    y = 1.0 - poly * jnp.exp(-ax * ax)
    return jnp.sign(x) * y


def _gelu(x):
    return x * 0.5 * (1.0 + _erf(x * jnp.float32(0.7071067811865476)))


def _mlp_kernel(x_ref, ids_ref, w1_ref, b1_ref, w2_ref, b2_ref, w3_ref, b3_ref,
                ds_ref, do_ref, a_ref, b_ref, u_ref, v_ref):
    S = x_ref.shape[0]
    E = w1_ref.shape[0]
    H = ds_ref.shape[1]
    x = x_ref[...]
    ids = ids_ref[...]
    eiota = jax.lax.broadcasted_iota(jnp.int32, (S, E), 1)
    onehot = (ids == eiota).astype(jnp.float32)

    def dot_t(a, w):
        return jax.lax.dot_general(a, w, (((1,), (1,)), ((), ())),
                                   preferred_element_type=jnp.float32)

    pb = jnp.zeros((S, H), jnp.float32)
    for e in range(E):
        h1 = _gelu(dot_t(x, w1_ref[e]) + b1_ref[e])
        h2 = _gelu(dot_t(h1, w2_ref[e]) + b2_ref[e])
        eo = dot_t(h2, w3_ref[e]) + b3_ref[e]
        pb = pb + onehot[:, e:e + 1] * eo

    counts = jnp.sum(onehot, axis=0, keepdims=True)
    inv_s = jnp.float32(1.0 / S)
    c = jnp.dot(counts, ds_ref[...], preferred_element_type=jnp.float32) \
        * (inv_s * jnp.float32(0.01))
    off = jnp.dot(counts, do_ref[...], preferred_element_type=jnp.float32) * inv_s
    eoff = jnp.exp(off)

    kk = jax.lax.broadcasted_iota(jnp.int32, (S, 1), 0).astype(jnp.float32) \
        - jnp.float32(S // 2)
    u = jnp.exp(-kk * c)
    v = jnp.exp(kk * c)
    pbo = pb * eoff
    a_ref[...] = pbo * u
    b_ref[...] = pbo * v
    u_ref[...] = u
    v_ref[...] = v


def _bias_kernel(a_ref, b_ref, u_ref, v_ref, o_ref, *, tile_i, tile_j):
    ib = pl.program_id(1)
    jb = pl.program_id(2)
    i0 = ib * tile_i
    j0 = jb * tile_j
    below = i0 >= j0 + tile_j
    above = j0 >= i0 + tile_i

    @pl.when(below)
    def _():
        o_ref[0] = a_ref[0] * v_ref[0]

    @pl.when(above)
    def _():
        o_ref[0] = b_ref[0] * u_ref[0]

    @pl.when(jnp.logical_not(jnp.logical_or(below, above)))
    def _():
        rows = jax.lax.broadcasted_iota(jnp.int32, (tile_i, tile_j), 0) + i0
        cols = jax.lax.broadcasted_iota(jnp.int32, (tile_i, tile_j), 1) + j0
        o_ref[0] = jnp.where(rows >= cols, a_ref[0] * v_ref[0],
                             b_ref[0] * u_ref[0])


def kernel(scalars, seq_len, inscription_ids, W1, b1, W2, b2, W3, b3,
           distance_scales, distance_offsets):
    del seq_len
    B, S, D = scalars.shape
    H = W3.shape[1]

    x = scalars.reshape(S, D)
    ids = inscription_ids.reshape(S, 1).astype(jnp.int32)

    sh = jax.ShapeDtypeStruct((S, H), jnp.float32)
    a2, b2_, u2, v2 = pl.pallas_call(
        _mlp_kernel,
        out_shape=(sh, sh, sh, sh),
    )(x, ids, W1, b1, W2, b2, W3, b3, distance_scales, distance_offsets)

    a3 = a2.T.reshape(H, S, 1)
    b3_ = b2_.T.reshape(H, S, 1)
    u3 = u2.T.reshape(H, 1, S)
    v3 = v2.T.reshape(H, 1, S)

    TILE_I = 1024
    TILE_J = S
    grid = (H, S // TILE_I, S // TILE_J)
    row_spec = pl.BlockSpec((1, TILE_I, 1), lambda h, i, j: (h, i, 0))
    col_spec = pl.BlockSpec((1, 1, TILE_J), lambda h, i, j: (h, 0, j))
    bias = pl.pallas_call(
        functools.partial(_bias_kernel, tile_i=TILE_I, tile_j=TILE_J),
        grid=grid,
        in_specs=[row_spec, row_spec, col_spec, col_spec],
        out_specs=pl.BlockSpec((1, TILE_I, TILE_J), lambda h, i, j: (h, i, j)),
        out_shape=jax.ShapeDtypeStruct((H, S, S), jnp.float32),
    )(a3, b3_, u3, v3)

    return bias.reshape(B, H, S, S)


# speedup vs baseline: 2.4191x; 1.0173x over previous
import functools

import jax
import jax.numpy as jnp
from jax.experimental import pallas as pl
from jax.experimental.pallas import tpu as pltpu


def _erf(x):
    p = jnp.float32(0.3275911)
    a1 = jnp.float32(0.254829592)
    a2 = jnp.float32(-0.284496736)
    a3 = jnp.float32(1.421413741)
    a4 = jnp.float32(-1.453152027)
    a5 = jnp.float32(1.061405429)
    ax = jnp.abs(x)
    t = 1.0 / (1.0 + p * ax)
    poly = t * (a1 + t * (a2 + t * (a3 + t * (a4 + t * a5))))
    y = 1.0 - poly * jnp.exp(-ax * ax)
    return jnp.sign(x) * y


def _gelu(x):
    return x * 0.5 * (1.0 + _erf(x * jnp.float32(0.7071067811865476)))


def _mlp_kernel(x_ref, ids_ref, w1_ref, b1_ref, w2_ref, b2_ref, w3_ref, b3_ref,
                ds_ref, do_ref, a_ref, b_ref, u_ref, v_ref):
    S = x_ref.shape[0]
    E = w1_ref.shape[0]
    H = ds_ref.shape[1]
    x = x_ref[...]
    ids = ids_ref[...]
    eiota = jax.lax.broadcasted_iota(jnp.int32, (S, E), 1)
    onehot = (ids == eiota).astype(jnp.float32)

    def dot_t(a, w):
        return jax.lax.dot_general(a, w, (((1,), (1,)), ((), ())),
                                   preferred_element_type=jnp.float32)

    pb = jnp.zeros((S, H), jnp.float32)
    for e in range(E):
        h1 = _gelu(dot_t(x, w1_ref[e]) + b1_ref[e])
        h2 = _gelu(dot_t(h1, w2_ref[e]) + b2_ref[e])
        eo = dot_t(h2, w3_ref[e]) + b3_ref[e]
        pb = pb + onehot[:, e:e + 1] * eo

    counts = jnp.sum(onehot, axis=0, keepdims=True)
    inv_s = jnp.float32(1.0 / S)
    c = jnp.dot(counts, ds_ref[...], preferred_element_type=jnp.float32) \
        * (inv_s * jnp.float32(0.01))
    off = jnp.dot(counts, do_ref[...], preferred_element_type=jnp.float32) * inv_s
    eoff = jnp.exp(off)

    kk = jax.lax.broadcasted_iota(jnp.int32, (S, 1), 0).astype(jnp.float32) \
        - jnp.float32(S // 2)
    u = jnp.exp(-kk * c)
    v = jnp.exp(kk * c)
    pbo = pb * eoff
    a_ref[...] = pbo * u
    b_ref[...] = pbo * v
    u_ref[...] = u
    v_ref[...] = v


def _bias_kernel(a_ref, b_ref, u_ref, v_ref, o_ref, *, tile_i, tile_j):
    ib = pl.program_id(1)
    jb = pl.program_id(2)
    i0 = ib * tile_i
    j0 = jb * tile_j
    below = i0 >= j0 + tile_j
    above = j0 >= i0 + tile_i

    @pl.when(below)
    def _():
        o_ref[0] = a_ref[0] * v_ref[0]

    @pl.when(above)
    def _():
        o_ref[0] = b_ref[0] * u_ref[0]

    @pl.when(jnp.logical_not(jnp.logical_or(below, above)))
    def _():
        rows = jax.lax.broadcasted_iota(jnp.int32, (tile_i, tile_j), 0) + i0
        cols = jax.lax.broadcasted_iota(jnp.int32, (tile_i, tile_j), 1) + j0
        o_ref[0] = jnp.where(rows >= cols, a_ref[0] * v_ref[0],
                             b_ref[0] * u_ref[0])


def kernel(scalars, seq_len, inscription_ids, W1, b1, W2, b2, W3, b3,
           distance_scales, distance_offsets):
    del seq_len
    B, S, D = scalars.shape
    H = W3.shape[1]

    x = scalars.reshape(S, D)
    ids = inscription_ids.reshape(S, 1).astype(jnp.int32)

    sh = jax.ShapeDtypeStruct((S, H), jnp.float32)
    a2, b2_, u2, v2 = pl.pallas_call(
        _mlp_kernel,
        out_shape=(sh, sh, sh, sh),
    )(x, ids, W1, b1, W2, b2, W3, b3, distance_scales, distance_offsets)

    a3 = a2.T.reshape(H, S, 1)
    b3_ = b2_.T.reshape(H, S, 1)
    u3 = u2.T.reshape(H, 1, S)
    v3 = v2.T.reshape(H, 1, S)

    TILE_I = 2048
    TILE_J = S
    grid = (H, S // TILE_I, S // TILE_J)
    row_spec = pl.BlockSpec((1, TILE_I, 1), lambda h, i, j: (h, i, 0))
    col_spec = pl.BlockSpec((1, 1, TILE_J), lambda h, i, j: (h, 0, j))
    bias = pl.pallas_call(
        functools.partial(_bias_kernel, tile_i=TILE_I, tile_j=TILE_J),
        grid=grid,
        in_specs=[row_spec, row_spec, col_spec, col_spec],
        out_specs=pl.BlockSpec((1, TILE_I, TILE_J), lambda h, i, j: (h, i, j)),
        out_shape=jax.ShapeDtypeStruct((H, S, S), jnp.float32),
    )(a3, b3_, u3, v3)

    return bias.reshape(B, H, S, S)


# speedup vs baseline: 3.4222x; 1.4147x over previous
import jax
import jax.numpy as jnp
from jax.experimental import pallas as pl
from jax.experimental.pallas import tpu as pltpu


def _erf(x):
    p = jnp.float32(0.3275911)
    a1 = jnp.float32(0.254829592)
    a2 = jnp.float32(-0.284496736)
    a3 = jnp.float32(1.421413741)
    a4 = jnp.float32(-1.453152027)
    a5 = jnp.float32(1.061405429)
    ax = jnp.abs(x)
    t = 1.0 / (1.0 + p * ax)
    poly = t * (a1 + t * (a2 + t * (a3 + t * (a4 + t * a5))))
    y = 1.0 - poly * jnp.exp(-ax * ax)
    return jnp.sign(x) * y


def _gelu(x):
    return x * 0.5 * (1.0 + _erf(x * jnp.float32(0.7071067811865476)))


def _fused_kernel(x_ref, ids_ref, w1_ref, b1_ref, w2_ref, b2_ref, w3_ref,
                  b3_ref, ds_ref, do_ref, o_ref, a_s, b_s, ut_s, vt_s):
    h = pl.program_id(0)
    S = x_ref.shape[0]
    E = w1_ref.shape[0]
    H = ds_ref.shape[1]

    @pl.when(h == 0)
    def _stage1():
        x = x_ref[...]
        ids = ids_ref[...]
        eiota = jax.lax.broadcasted_iota(jnp.int32, (S, E), 1)
        onehot = (ids == eiota).astype(jnp.float32)

        def dot_t(a, w):
            return jax.lax.dot_general(a, w, (((1,), (1,)), ((), ())),
                                       preferred_element_type=jnp.float32)

        pb = jnp.zeros((S, H), jnp.float32)
        for e in range(E):
            h1 = _gelu(dot_t(x, w1_ref[e]) + b1_ref[e])
            h2 = _gelu(dot_t(h1, w2_ref[e]) + b2_ref[e])
            eo = dot_t(h2, w3_ref[e]) + b3_ref[e]
            pb = pb + onehot[:, e:e + 1] * eo

        counts = jnp.sum(onehot, axis=0, keepdims=True)
        inv_s = jnp.float32(1.0 / S)
        c = jnp.dot(counts, ds_ref[...], preferred_element_type=jnp.float32) \
            * (inv_s * jnp.float32(0.01))
        off = jnp.dot(counts, do_ref[...], preferred_element_type=jnp.float32) * inv_s

        kk = jax.lax.broadcasted_iota(jnp.int32, (S, 1), 0).astype(jnp.float32) \
            - jnp.float32(S // 2)
        u = jnp.exp(-kk * c)
        v = jnp.exp(kk * c)
        pbo = pb * jnp.exp(off)
        a_s[...] = pbo * u
        b_s[...] = pbo * v

        hiota = jax.lax.broadcasted_iota(jnp.int32, (H, H), 1)
        hsel = (hiota == jax.lax.broadcasted_iota(jnp.int32, (H, H), 0))
        eyeh = hsel.astype(jnp.float32)
        cT = jnp.sum(jnp.broadcast_to(c, (H, H)) * eyeh, axis=1, keepdims=True)
        kl = jax.lax.broadcasted_iota(jnp.int32, (1, S), 1).astype(jnp.float32) \
            - jnp.float32(S // 2)
        ut_s[...] = jnp.exp(-kl * cT)
        vt_s[...] = jnp.exp(kl * cT)

    lane_h = jax.lax.broadcasted_iota(jnp.int32, (1, a_s.shape[1]), 1) == h
    onel = lane_h.astype(jnp.float32)
    a = jnp.sum(a_s[...] * onel, axis=1, keepdims=True)
    b = jnp.sum(b_s[...] * onel, axis=1, keepdims=True)
    sub_h = jax.lax.broadcasted_iota(jnp.int32, (ut_s.shape[0], 1), 0) == h
    ones = sub_h.astype(jnp.float32)
    u = jnp.sum(ut_s[...] * ones, axis=0, keepdims=True)
    v = jnp.sum(vt_s[...] * ones, axis=0, keepdims=True)
    rows = jax.lax.broadcasted_iota(jnp.int32, (S, S), 0)
    cols = jax.lax.broadcasted_iota(jnp.int32, (S, S), 1)
    o_ref[0] = jnp.where(rows >= cols, a * v, b * u)


def kernel(scalars, seq_len, inscription_ids, W1, b1, W2, b2, W3, b3,
           distance_scales, distance_offsets):
    del seq_len
    B, S, D = scalars.shape
    E, HID, _ = W1.shape
    H = W3.shape[1]

    x = scalars.reshape(S, D)
    ids = inscription_ids.reshape(S, 1).astype(jnp.int32)

    def whole(shape):
        return pl.BlockSpec(shape, lambda h: (0,) * len(shape))

    bias = pl.pallas_call(
        _fused_kernel,
        grid=(H,),
        in_specs=[
            whole((S, D)), whole((S, 1)),
            whole((E, HID, D)), whole((E, HID)),
            whole((E, HID, HID)), whole((E, HID)),
            whole((E, H, HID)), whole((E, H)),
            whole((E, H)), whole((E, H)),
        ],
        out_specs=pl.BlockSpec((1, S, S), lambda h: (h, 0, 0)),
        out_shape=jax.ShapeDtypeStruct((H, S, S), jnp.float32),
        scratch_shapes=[
            pltpu.VMEM((S, H), jnp.float32),
            pltpu.VMEM((S, H), jnp.float32),
            pltpu.VMEM((H, S), jnp.float32),
            pltpu.VMEM((H, S), jnp.float32),
        ],
    )(x, ids, W1, b1, W2, b2, W3, b3, distance_scales, distance_offsets)

    return bias.reshape(B, H, S, S)
